# baseline (speedup 1.0000x reference)
import functools

import jax
import jax.numpy as jnp
from jax import lax
from jax.experimental import pallas as pl
from jax.experimental.pallas import tpu as pltpu
from jax.experimental.pallas import tpu_sc as plsc

B, D_IN, T = 8, 1024, 1024
K, D_CODE = 8192, 256
KT = 1024
NKT = K // KT
EPS = 1e-12


def _project_body(z_ref, in_w_ref, in_b_ref, zeT_ref):
    zeT_ref[0] = (jnp.dot(in_w_ref[...], z_ref[0],
                          preferred_element_type=jnp.float32)
                  + in_b_ref[...]).T


def _project(z, in_w, in_b):
    return pl.pallas_call(
        _project_body,
        grid=(B,),
        in_specs=[
            pl.BlockSpec((1, D_IN, T), lambda b: (b, 0, 0)),
            pl.BlockSpec((D_CODE, D_IN), lambda b: (0, 0)),
            pl.BlockSpec((D_CODE, 1), lambda b: (0, 0)),
        ],
        out_specs=pl.BlockSpec((1, T, D_CODE), lambda b: (b, 0, 0)),
        out_shape=jax.ShapeDtypeStruct((B, T, D_CODE), jnp.float32),
    )(z, in_w, in_b.reshape(D_CODE, 1))


def _score_body(zeT_ref, cb_ref, idx_ref, zen_ref, esq_ref, best_ref, bidx_ref):
    k = pl.program_id(1)

    @pl.when(k == 0)
    def _init():
        zeT = zeT_ref[0]
        n = jnp.sqrt(jnp.sum(zeT * zeT, axis=1, keepdims=True))
        enc_n = zeT / jnp.maximum(n, EPS)
        zen_ref[...] = enc_n
        esq_ref[...] = jnp.sum(enc_n * enc_n, axis=1, keepdims=True)
        best_ref[...] = jnp.full((T, 1), -jnp.inf, dtype=jnp.float32)
        bidx_ref[...] = jnp.zeros((T, 1), dtype=jnp.int32)

    cb = cb_ref[...]
    cb_n = cb / jnp.maximum(
        jnp.sqrt(jnp.sum(cb * cb, axis=1, keepdims=True)), EPS)
    cb_sq = jnp.sum(cb_n * cb_n, axis=1, keepdims=True)
    en = zen_ref[...]
    en_u = en.astype(jnp.bfloat16).astype(jnp.float32)
    en_r = en - en_u
    cbT = cb_n.T.astype(jnp.bfloat16)
    dot = (jnp.dot(en_u.astype(jnp.bfloat16), cbT,
                   preferred_element_type=jnp.float32)
           + jnp.dot(en_r.astype(jnp.bfloat16), cbT,
                     preferred_element_type=jnp.float32))
    scores = -((esq_ref[...] - 2.0 * dot) + cb_sq.T)
    rowmax = jnp.max(scores, axis=1, keepdims=True)
    cols = lax.broadcasted_iota(jnp.int32, (T, KT), 1) + k * KT
    cand = jnp.min(jnp.where(scores == rowmax, cols, jnp.int32(2**31 - 1)),
                   axis=1, keepdims=True)
    better = rowmax > best_ref[...]
    bidx_ref[...] = jnp.where(better, cand, bidx_ref[...])
    best_ref[...] = jnp.where(better, rowmax, best_ref[...])

    @pl.when(k == NKT - 1)
    def _fin():
        idx_ref[0] = bidx_ref[...]


def _encode(z, in_w, in_b, codebook):
    zeT = _project(z, in_w, in_b)
    idx3 = pl.pallas_call(
        _score_body,
        grid=(B, NKT),
        in_specs=[
            pl.BlockSpec((1, T, D_CODE), lambda b, k: (b, 0, 0)),
            pl.BlockSpec((KT, D_CODE), lambda b, k: (k, 0)),
        ],
        out_specs=pl.BlockSpec((1, T, 1), lambda b, k: (b, 0, 0)),
        out_shape=jax.ShapeDtypeStruct((B, T, 1), jnp.int32),
        scratch_shapes=[
            pltpu.VMEM((T, D_CODE), jnp.float32),
            pltpu.VMEM((T, 1), jnp.float32),
            pltpu.VMEM((T, 1), jnp.float32),
            pltpu.VMEM((T, 1), jnp.int32),
        ],
        compiler_params=pltpu.CompilerParams(
            dimension_semantics=("arbitrary", "arbitrary")),
    )(zeT, codebook)
    z_e = jnp.transpose(zeT, (0, 2, 1))
    return z_e, idx3.reshape(B, T)


_NC, _NS = 2, 16
_NW = _NC * _NS
_ROWS_PER_W = (B * T) // _NW


def _gather_body(idx_hbm, table_hbm, out_hbm, idx_v, rows_v, sem):
    wid = lax.axis_index("s") * _NC + lax.axis_index("c")
    base = wid * _ROWS_PER_W
    pltpu.sync_copy(idx_hbm.at[pl.ds(base, _ROWS_PER_W)], idx_v)
    pltpu.async_copy(table_hbm.at[idx_v], rows_v, sem).wait()
    pltpu.sync_copy(rows_v, out_hbm.at[pl.ds(base, _ROWS_PER_W)])


def _gather(idx_flat, table):
    return pl.kernel(
        _gather_body,
        out_type=jax.ShapeDtypeStruct((B * T, D_CODE), jnp.float32),
        mesh=plsc.VectorSubcoreMesh(core_axis_name="c", subcore_axis_name="s"),
        scratch_types=[
            pltpu.VMEM((_ROWS_PER_W,), jnp.int32),
            pltpu.VMEM((_ROWS_PER_W, D_CODE), jnp.float32),
            pltpu.SemaphoreType.DMA,
        ],
    )(idx_flat, table)


def _decode_body(zq_ref, z_e_ref, out_w_ref, out_b_ref, out_ref, loss_ref):
    zqt = zq_ref[0].T
    out_ref[0] = jnp.dot(out_w_ref[...], zqt,
                         preferred_element_type=jnp.float32) + out_b_ref[...]
    diff = z_e_ref[0] - zqt
    loss_ref[0] = jnp.sum(diff * diff, keepdims=True) * (1.0 / (D_CODE * T))


def _decode(z_q, z_e, out_w, out_b):
    out, loss = pl.pallas_call(
        _decode_body,
        grid=(B,),
        in_specs=[
            pl.BlockSpec((1, T, D_CODE), lambda b: (b, 0, 0)),
            pl.BlockSpec((1, D_CODE, T), lambda b: (b, 0, 0)),
            pl.BlockSpec((D_IN, D_CODE), lambda b: (0, 0)),
            pl.BlockSpec((D_IN, 1), lambda b: (0, 0)),
        ],
        out_specs=[
            pl.BlockSpec((1, D_IN, T), lambda b: (b, 0, 0)),
            pl.BlockSpec((1, 1, 1), lambda b: (b, 0, 0)),
        ],
        out_shape=[
            jax.ShapeDtypeStruct((B, D_IN, T), jnp.float32),
            jax.ShapeDtypeStruct((B, 1, 1), jnp.float32),
        ],
    )(z_q, z_e, out_w, out_b.reshape(D_IN, 1))
    return out, loss.reshape(B)


def kernel(z, in_w, in_b, codebook, out_w, out_b):
    z_e, indices = _encode(z, in_w, in_b, codebook)
    z_q_flat = _gather(indices.reshape(B * T), codebook)
    z_q = z_q_flat.reshape(B, T, D_CODE)
    out, loss = _decode(z_q, z_e, out_w, out_b)
    return (out, loss, loss, indices, z_e)
